# Initial kernel scaffold; baseline (speedup 1.0000x reference)
import jax
import jax.numpy as jnp
from jax.experimental import pallas as pl


def kernel(n_coords, ca_coords, c_coords, W0, b0, Wl, bl, Wsl, bsl, gamma, beta):
    raise NotImplementedError("write your pallas kernel here")


# speedup vs baseline: 4.6131x; 4.6131x over previous
import functools

import jax
import jax.numpy as jnp
from jax import lax
from jax.experimental import pallas as pl
from jax.experimental.pallas import tpu as pltpu
from jax.experimental.pallas import tpu_sc as plsc

B = 4
L = 2048
N = B * L
H = 512
NUM_REL = 7
KNN = 10
LAYERS = 4

R = 256
NBLK = N // H

NC = 2
NS = 16
NW = NC * NS
NPW = N // NW
C = 8
NCHUNK = NPW // C


def _knn_body(rows_ref, cols_ref, key_ref):
    b = pl.program_id(0)
    rb = pl.program_id(1)
    r0 = rb * R
    a = rows_ref[...]
    cm = cols_ref[0, :, :]
    d2 = (a[:, 0:1] - cm[0:1, :]) ** 2
    d2 = d2 + (a[:, 1:2] - cm[1:2, :]) ** 2
    d2 = d2 + (a[:, 2:3] - cm[2:3, :]) ** 2
    col = lax.broadcasted_iota(jnp.int32, (R, L), 1)
    row2 = r0 + lax.broadcasted_iota(jnp.int32, (R, 1), 0)
    d2 = jnp.where(col == row2, d2 + 1e9, d2)

    kiota = lax.broadcasted_iota(jnp.int32, (R, 16), 1)
    kacc = jnp.zeros((R, 16), jnp.int32)
    big = jnp.float32(2e9)
    for k in range(KNN):
        m = jnp.min(d2, axis=1, keepdims=True)
        idx = jnp.min(jnp.where(d2 == m, col, N), axis=1,
                      keepdims=True)
        rel = jnp.clip(idx - row2, -3, 3) + 3
        key = rel * N + b * L + idx
        kacc = jnp.where(kiota == k, key, kacc)
        d2 = jnp.where(col == idx, big, d2)
    key_ref[0, :, :] = kacc


def _knn_keys(cap, caT):
    return pl.pallas_call(
        _knn_body,
        grid=(B, L // R),
        in_specs=[
            pl.BlockSpec((R, 8), lambda b, rb: (b * (L // R) + rb, 0)),
            pl.BlockSpec((1, 8, L), lambda b, rb: (b, 0, 0)),
        ],
        out_specs=pl.BlockSpec((1, R, 16), lambda b, rb: (b * (L // R) + rb, 0, 0)),
        out_shape=jax.ShapeDtypeStruct((B * (L // R), R, 16), jnp.int32),
    )(cap, caT)


def _proj_body(pos_ref, w_ref, b_ref, out_ref):
    out_ref[...] = (
        jnp.dot(pos_ref[...], w_ref[...], preferred_element_type=jnp.float32)
        + b_ref[...]
    )


def _proj(posp, w0p, b0r):
    return pl.pallas_call(
        _proj_body,
        grid=(NBLK,),
        in_specs=[
            pl.BlockSpec((H, 8), lambda i: (i, 0)),
            pl.BlockSpec((8, H), lambda i: (0, 0)),
            pl.BlockSpec((1, H), lambda i: (0, 0)),
        ],
        out_specs=pl.BlockSpec((H, H), lambda i: (i, 0)),
        out_shape=jax.ShapeDtypeStruct((N, H), jnp.float32),
    )(posp, w0p, b0r)


def _hall_body(h_ref, w_ref, out_ref):
    out_ref[0] = jnp.dot(h_ref[...], w_ref[0],
                         preferred_element_type=jnp.float32)


def _hall(h, wl3):
    return pl.pallas_call(
        _hall_body,
        grid=(NUM_REL, NBLK),
        in_specs=[
            pl.BlockSpec((H, H), lambda r, nb: (nb, 0)),
            pl.BlockSpec((1, H, H), lambda r, nb: (r, 0, 0)),
        ],
        out_specs=pl.BlockSpec((1, H, H), lambda r, nb: (r, nb, 0)),
        out_shape=jax.ShapeDtypeStruct((NUM_REL, N, H), jnp.float32),
    )(h, wl3)


def _sc_gather_body(keys_hbm, table_hbm, out_hbm, idx_v, buf_v, acc_v, sem):
    wid = lax.axis_index("s") * NC + lax.axis_index("c")
    base = wid * NPW
    pltpu.sync_copy(keys_hbm.at[pl.ds(wid * NPW * KNN, NPW * KNN)], idx_v)

    def chunk(g, carry):
        pltpu.async_copy(
            table_hbm.at[idx_v.at[pl.ds(g * C * KNN, C * KNN)]], buf_v, sem
        ).wait()

        def node(n, c2):
            for j in range(H // 16):
                sl = pl.ds(j * 16, 16)
                s = buf_v[n * KNN, sl]
                for k in range(1, KNN):
                    s = s + buf_v[n * KNN + k, sl]
                acc_v[n, sl] = s
            return c2

        lax.fori_loop(0, C, node, 0)
        pltpu.sync_copy(acc_v, out_hbm.at[pl.ds(base + g * C, C)])
        return carry

    lax.fori_loop(0, NCHUNK, chunk, 0)


@functools.cache
def _make_sc_gather():
    return pl.kernel(
        _sc_gather_body,
        mesh=plsc.VectorSubcoreMesh(core_axis_name="c", subcore_axis_name="s"),
        out_type=jax.ShapeDtypeStruct((N, H), jnp.float32),
        scratch_types=[
            pltpu.VMEM((NPW * KNN,), jnp.int32),
            pltpu.VMEM((C * KNN, H), jnp.float32),
            pltpu.VMEM((C, H), jnp.float32),
            pltpu.SemaphoreType.DMA,
        ],
    )


def _sc_gather(keys, table):
    return _make_sc_gather()(keys, table)


def _stats_body(msg_ref, h_ref, w_ref, bias_ref, pre_ref, ps_ref, pq_ref):
    pre = (
        msg_ref[...]
        + jnp.dot(h_ref[...], w_ref[...], preferred_element_type=jnp.float32)
        + bias_ref[...]
    )
    pre_ref[...] = pre
    ps_ref[0] = jnp.sum(pre, axis=0, keepdims=True)
    pq_ref[0] = jnp.sum(pre * pre, axis=0, keepdims=True)


def _stats(msg, h, wsl, bias):
    return pl.pallas_call(
        _stats_body,
        grid=(NBLK,),
        in_specs=[
            pl.BlockSpec((H, H), lambda i: (i, 0)),
            pl.BlockSpec((H, H), lambda i: (i, 0)),
            pl.BlockSpec((H, H), lambda i: (0, 0)),
            pl.BlockSpec((1, H), lambda i: (0, 0)),
        ],
        out_specs=[
            pl.BlockSpec((H, H), lambda i: (i, 0)),
            pl.BlockSpec((1, 1, H), lambda i: (i, 0, 0)),
            pl.BlockSpec((1, 1, H), lambda i: (i, 0, 0)),
        ],
        out_shape=[
            jax.ShapeDtypeStruct((N, H), jnp.float32),
            jax.ShapeDtypeStruct((NBLK, 1, H), jnp.float32),
            jax.ShapeDtypeStruct((NBLK, 1, H), jnp.float32),
        ],
    )(msg, h, wsl, bias)


def _norm_body(pre_ref, h_ref, ps_ref, pq_ref, g_ref, b_ref, out_ref):
    s = jnp.sum(ps_ref[:, 0, :], axis=0, keepdims=True)
    q = jnp.sum(pq_ref[:, 0, :], axis=0, keepdims=True)
    mean = s * (1.0 / N)
    var = q * (1.0 / N) - mean * mean
    inv = lax.rsqrt(var + 1e-5)
    x = (pre_ref[...] - mean) * inv * g_ref[...] + b_ref[...]
    out_ref[...] = jnp.maximum(x, 0.0) + h_ref[...]


def _norm(pre, h, ps, pq, gamma, beta):
    return pl.pallas_call(
        _norm_body,
        grid=(NBLK,),
        in_specs=[
            pl.BlockSpec((H, H), lambda i: (i, 0)),
            pl.BlockSpec((H, H), lambda i: (i, 0)),
            pl.BlockSpec((NBLK, 1, H), lambda i: (0, 0, 0)),
            pl.BlockSpec((NBLK, 1, H), lambda i: (0, 0, 0)),
            pl.BlockSpec((1, H), lambda i: (0, 0)),
            pl.BlockSpec((1, H), lambda i: (0, 0)),
        ],
        out_specs=pl.BlockSpec((H, H), lambda i: (i, 0)),
        out_shape=jax.ShapeDtypeStruct((N, H), jnp.float32),
    )(pre, h, ps, pq, gamma, beta)


def kernel(n_coords, ca_coords, c_coords, W0, b0, Wl, bl, Wsl, bsl, gamma, beta):
    ca = ca_coords.reshape(N, 3)
    cap = jnp.pad(ca, ((0, 0), (0, 5)))
    caT = jnp.pad(jnp.transpose(ca_coords, (0, 2, 1)),
                  ((0, 0), (0, 5), (0, 0)))

    keys4 = _knn_keys(cap, caT)
    keys = keys4.reshape(N, 16)[:, :KNN].reshape(-1)

    w0p = jnp.pad(W0, ((0, 5), (0, 0)))
    h = _proj(cap, w0p, b0.reshape(1, H))

    for i in range(LAYERS):
        wl3 = Wl[i].reshape(NUM_REL, H, H)
        hall = _hall(h, wl3).reshape(NUM_REL * N, H)
        msg = _sc_gather(keys, hall)
        bias = (bl[i] + bsl[i]).reshape(1, H)
        pre, ps, pq = _stats(msg, h, Wsl[i], bias)
        h = _norm(pre, h, ps, pq, gamma[i].reshape(1, H), beta[i].reshape(1, H))

    return h.reshape(B, L, H)


# speedup vs baseline: 5.0533x; 1.0954x over previous
import functools

import jax
import jax.numpy as jnp
from jax import lax
from jax.experimental import pallas as pl
from jax.experimental.pallas import tpu as pltpu
from jax.experimental.pallas import tpu_sc as plsc

B = 4
L = 2048
N = B * L
H = 512
NUM_REL = 7
KNN = 10
LAYERS = 4

R = 256
NBLK = N // H

NC = 2
NS = 16
NW = NC * NS
NPW = N // NW
C = 8
NCHUNK = NPW // C


def _knn_body(rows_ref, cols_ref, key_ref):
    b = pl.program_id(0)
    rb = pl.program_id(1)
    r0 = rb * R
    a = rows_ref[...]
    cm = cols_ref[0, :, :]
    d2 = (a[:, 0:1] - cm[0:1, :]) ** 2
    d2 = d2 + (a[:, 1:2] - cm[1:2, :]) ** 2
    d2 = d2 + (a[:, 2:3] - cm[2:3, :]) ** 2
    col = lax.broadcasted_iota(jnp.int32, (R, L), 1)
    row2 = r0 + lax.broadcasted_iota(jnp.int32, (R, 1), 0)
    d2 = jnp.where(col == row2, d2 + 1e9, d2)

    kiota = lax.broadcasted_iota(jnp.int32, (R, 16), 1)
    kacc = jnp.zeros((R, 16), jnp.int32)
    big = jnp.float32(2e9)
    for k in range(KNN):
        m = jnp.min(d2, axis=1, keepdims=True)
        idx = jnp.min(jnp.where(d2 == m, col, N), axis=1,
                      keepdims=True)
        rel = jnp.clip(idx - row2, -3, 3) + 3
        key = rel * N + b * L + idx
        kacc = jnp.where(kiota == k, key, kacc)
        d2 = jnp.where(col == idx, big, d2)
    key_ref[0, :, :] = kacc


def _knn_keys(cap, caT):
    return pl.pallas_call(
        _knn_body,
        grid=(B, L // R),
        in_specs=[
            pl.BlockSpec((R, 8), lambda b, rb: (b * (L // R) + rb, 0)),
            pl.BlockSpec((1, 8, L), lambda b, rb: (b, 0, 0)),
        ],
        out_specs=pl.BlockSpec((1, R, 16), lambda b, rb: (b * (L // R) + rb, 0, 0)),
        out_shape=jax.ShapeDtypeStruct((B * (L // R), R, 16), jnp.int32),
    )(cap, caT)


def _proj_body(pos_ref, w_ref, b_ref, out_ref):
    out_ref[...] = (
        jnp.dot(pos_ref[...], w_ref[...], preferred_element_type=jnp.float32)
        + b_ref[...]
    )


def _proj(posp, w0p, b0r):
    return pl.pallas_call(
        _proj_body,
        grid=(NBLK,),
        in_specs=[
            pl.BlockSpec((H, 8), lambda i: (i, 0)),
            pl.BlockSpec((8, H), lambda i: (0, 0)),
            pl.BlockSpec((1, H), lambda i: (0, 0)),
        ],
        out_specs=pl.BlockSpec((H, H), lambda i: (i, 0)),
        out_shape=jax.ShapeDtypeStruct((N, H), jnp.float32),
    )(posp, w0p, b0r)


def _hall_body(h_ref, w_ref, out_ref):
    out_ref[0] = jnp.dot(h_ref[...], w_ref[0],
                         preferred_element_type=jnp.float32)


def _hall(h, wl3):
    return pl.pallas_call(
        _hall_body,
        grid=(NUM_REL + 1, NBLK),
        in_specs=[
            pl.BlockSpec((H, H), lambda r, nb: (nb, 0)),
            pl.BlockSpec((1, H, H), lambda r, nb: (r, 0, 0)),
        ],
        out_specs=pl.BlockSpec((1, H, H), lambda r, nb: (r, nb, 0)),
        out_shape=jax.ShapeDtypeStruct((NUM_REL + 1, N, H), jnp.float32),
    )(h, wl3)


def _sc_gather_body(keys_hbm, table_hbm, out_hbm, idx_v, buf0, buf1, acc_v,
                    sem0, sem1):
    wid = lax.axis_index("s") * NC + lax.axis_index("c")
    base = wid * NPW
    pltpu.sync_copy(keys_hbm.at[pl.ds(wid * NPW * KNN, NPW * KNN)], idx_v)

    def start(g, buf, sem):
        return pltpu.async_copy(
            table_hbm.at[idx_v.at[pl.ds(g * C * KNN, C * KNN)]], buf, sem
        )

    def wait(buf, sem):
        pltpu.make_async_copy(
            table_hbm.at[idx_v.at[pl.ds(0, C * KNN)]], buf, sem
        ).wait()

    def segsum(buf, g):
        def node(n, c2):
            for j in range(H // 16):
                sl = pl.ds(j * 16, 16)
                s = buf[n * KNN, sl]
                for k in range(1, KNN):
                    s = s + buf[n * KNN + k, sl]
                acc_v[n, sl] = s
            return c2

        lax.fori_loop(0, C, node, 0)
        pltpu.sync_copy(acc_v, out_hbm.at[pl.ds(base + g * C, C)])

    start(0, buf0, sem0)

    def pair(gg, carry):
        g0 = gg * 2
        start(g0 + 1, buf1, sem1)
        wait(buf0, sem0)
        segsum(buf0, g0)

        @pl.when(gg < NCHUNK // 2 - 1)
        def _():
            start(g0 + 2, buf0, sem0)

        wait(buf1, sem1)
        segsum(buf1, g0 + 1)
        return carry

    lax.fori_loop(0, NCHUNK // 2, pair, 0)


@functools.cache
def _make_sc_gather():
    return pl.kernel(
        _sc_gather_body,
        mesh=plsc.VectorSubcoreMesh(core_axis_name="c", subcore_axis_name="s"),
        out_type=jax.ShapeDtypeStruct((N, H), jnp.float32),
        scratch_types=[
            pltpu.VMEM((NPW * KNN,), jnp.int32),
            pltpu.VMEM((C * KNN, H), jnp.float32),
            pltpu.VMEM((C * KNN, H), jnp.float32),
            pltpu.VMEM((C, H), jnp.float32),
            pltpu.SemaphoreType.DMA,
            pltpu.SemaphoreType.DMA,
        ],
    )


def _sc_gather(keys, table):
    return _make_sc_gather()(keys, table)


def _stats_body(msg_ref, s_ref, bias_ref, pre_ref, ps_ref, pq_ref):
    pre = msg_ref[...] + s_ref[...] + bias_ref[...]
    pre_ref[...] = pre
    ps_ref[0] = jnp.sum(pre, axis=0, keepdims=True)
    pq_ref[0] = jnp.sum(pre * pre, axis=0, keepdims=True)


def _stats(msg, hall2d, bias):
    return pl.pallas_call(
        _stats_body,
        grid=(NBLK,),
        in_specs=[
            pl.BlockSpec((H, H), lambda i: (i, 0)),
            pl.BlockSpec((H, H), lambda i: (NUM_REL * NBLK + i, 0)),
            pl.BlockSpec((1, H), lambda i: (0, 0)),
        ],
        out_specs=[
            pl.BlockSpec((H, H), lambda i: (i, 0)),
            pl.BlockSpec((1, 1, H), lambda i: (i, 0, 0)),
            pl.BlockSpec((1, 1, H), lambda i: (i, 0, 0)),
        ],
        out_shape=[
            jax.ShapeDtypeStruct((N, H), jnp.float32),
            jax.ShapeDtypeStruct((NBLK, 1, H), jnp.float32),
            jax.ShapeDtypeStruct((NBLK, 1, H), jnp.float32),
        ],
    )(msg, hall2d, bias)


def _norm_body(pre_ref, h_ref, ps_ref, pq_ref, g_ref, b_ref, out_ref):
    s = jnp.sum(ps_ref[:, 0, :], axis=0, keepdims=True)
    q = jnp.sum(pq_ref[:, 0, :], axis=0, keepdims=True)
    mean = s * (1.0 / N)
    var = q * (1.0 / N) - mean * mean
    inv = lax.rsqrt(var + 1e-5)
    x = (pre_ref[...] - mean) * inv * g_ref[...] + b_ref[...]
    out_ref[...] = jnp.maximum(x, 0.0) + h_ref[...]


def _norm(pre, h, ps, pq, gamma, beta):
    return pl.pallas_call(
        _norm_body,
        grid=(NBLK,),
        in_specs=[
            pl.BlockSpec((H, H), lambda i: (i, 0)),
            pl.BlockSpec((H, H), lambda i: (i, 0)),
            pl.BlockSpec((NBLK, 1, H), lambda i: (0, 0, 0)),
            pl.BlockSpec((NBLK, 1, H), lambda i: (0, 0, 0)),
            pl.BlockSpec((1, H), lambda i: (0, 0)),
            pl.BlockSpec((1, H), lambda i: (0, 0)),
        ],
        out_specs=pl.BlockSpec((H, H), lambda i: (i, 0)),
        out_shape=jax.ShapeDtypeStruct((N, H), jnp.float32),
    )(pre, h, ps, pq, gamma, beta)


def kernel(n_coords, ca_coords, c_coords, W0, b0, Wl, bl, Wsl, bsl, gamma, beta):
    ca = ca_coords.reshape(N, 3)
    cap = jnp.pad(ca, ((0, 0), (0, 5)))
    caT = jnp.pad(jnp.transpose(ca_coords, (0, 2, 1)),
                  ((0, 0), (0, 5), (0, 0)))

    keys4 = _knn_keys(cap, caT)
    keys = keys4.reshape(N, 16)[:, :KNN].reshape(-1)

    w0p = jnp.pad(W0, ((0, 5), (0, 0)))
    h = _proj(cap, w0p, b0.reshape(1, H))

    for i in range(LAYERS):
        wl3 = jnp.concatenate(
            [Wl[i].reshape(NUM_REL, H, H), Wsl[i][None]], axis=0)
        hall2d = _hall(h, wl3).reshape((NUM_REL + 1) * N, H)
        msg = _sc_gather(keys, hall2d)
        bias = (bl[i] + bsl[i]).reshape(1, H)
        pre, ps, pq = _stats(msg, hall2d, bias)
        h = _norm(pre, h, ps, pq, gamma[i].reshape(1, H), beta[i].reshape(1, H))

    return h.reshape(B, L, H)


# speedup vs baseline: 6.7218x; 1.3302x over previous
import functools

import jax
import jax.numpy as jnp
from jax import lax
from jax.experimental import pallas as pl
from jax.experimental.pallas import tpu as pltpu
from jax.experimental.pallas import tpu_sc as plsc

B = 4
L = 2048
N = B * L
H = 512
NUM_REL = 7
KNN = 10
LAYERS = 4

R = 256
NBLK = N // H

NC = 2
NS = 16
NW = NC * NS
NPW = N // NW
C = 8
NCHUNK = NPW // C


def _knn_body(rows_ref, cols_ref, key_ref):
    b = pl.program_id(0)
    rb = pl.program_id(1)
    r0 = rb * R
    a = rows_ref[...]
    cm = cols_ref[0, :, :]
    d2 = (a[:, 0:1] - cm[0:1, :]) ** 2
    d2 = d2 + (a[:, 1:2] - cm[1:2, :]) ** 2
    d2 = d2 + (a[:, 2:3] - cm[2:3, :]) ** 2
    col = lax.broadcasted_iota(jnp.int32, (R, L), 1)
    row2 = r0 + lax.broadcasted_iota(jnp.int32, (R, 1), 0)
    d2 = jnp.where(col == row2, d2 + 1e9, d2)

    kiota = lax.broadcasted_iota(jnp.int32, (R, 16), 1)
    kacc = jnp.zeros((R, 16), jnp.int32)
    big = jnp.float32(2e9)
    for k in range(KNN):
        m = jnp.min(d2, axis=1, keepdims=True)
        idx = jnp.min(jnp.where(d2 == m, col, N), axis=1,
                      keepdims=True)
        rel = jnp.clip(idx - row2, -3, 3) + 3
        key = rel * N + b * L + idx
        kacc = jnp.where(kiota == k, key, kacc)
        d2 = jnp.where(col == idx, big, d2)
    key_ref[0, :, :] = kacc


def _knn_keys(cap, caT):
    return pl.pallas_call(
        _knn_body,
        grid=(B, L // R),
        in_specs=[
            pl.BlockSpec((R, 8), lambda b, rb: (b * (L // R) + rb, 0)),
            pl.BlockSpec((1, 8, L), lambda b, rb: (b, 0, 0)),
        ],
        out_specs=pl.BlockSpec((1, R, 16), lambda b, rb: (b * (L // R) + rb, 0, 0)),
        out_shape=jax.ShapeDtypeStruct((B * (L // R), R, 16), jnp.int32),
    )(cap, caT)


def _proj_body(pos_ref, w_ref, b_ref, out_ref, outb_ref):
    h0 = (
        jnp.dot(pos_ref[...], w_ref[...], preferred_element_type=jnp.float32)
        + b_ref[...]
    )
    out_ref[...] = h0
    outb_ref[...] = h0.astype(jnp.bfloat16)


def _proj(posp, w0p, b0r):
    return pl.pallas_call(
        _proj_body,
        grid=(NBLK,),
        in_specs=[
            pl.BlockSpec((H, 8), lambda i: (i, 0)),
            pl.BlockSpec((8, H), lambda i: (0, 0)),
            pl.BlockSpec((1, H), lambda i: (0, 0)),
        ],
        out_specs=[
            pl.BlockSpec((H, H), lambda i: (i, 0)),
            pl.BlockSpec((H, H), lambda i: (i, 0)),
        ],
        out_shape=[
            jax.ShapeDtypeStruct((N, H), jnp.float32),
            jax.ShapeDtypeStruct((N, H), jnp.bfloat16),
        ],
    )(posp, w0p, b0r)


def _hall_body(hb_ref, w_ref, out_ref):
    out_ref[0] = jnp.dot(hb_ref[...], w_ref[0],
                         preferred_element_type=jnp.float32)


def _hall(hb, wl3b):
    return pl.pallas_call(
        _hall_body,
        grid=(NUM_REL + 1,),
        in_specs=[
            pl.BlockSpec((N, H), lambda r: (0, 0)),
            pl.BlockSpec((1, H, H), lambda r: (r, 0, 0)),
        ],
        out_specs=pl.BlockSpec((1, N, H), lambda r: (r, 0, 0)),
        out_shape=jax.ShapeDtypeStruct((NUM_REL + 1, N, H), jnp.float32),
    )(hb, wl3b)


def _sc_gather_body(keys_hbm, table_hbm, out_hbm, idx_v, buf0, buf1, acc_v,
                    sem0, sem1):
    wid = lax.axis_index("s") * NC + lax.axis_index("c")
    base = wid * NPW
    pltpu.sync_copy(keys_hbm.at[pl.ds(wid * NPW * KNN, NPW * KNN)], idx_v)

    def start(g, buf, sem):
        return pltpu.async_copy(
            table_hbm.at[idx_v.at[pl.ds(g * C * KNN, C * KNN)]], buf, sem
        )

    def wait(buf, sem):
        pltpu.make_async_copy(
            table_hbm.at[idx_v.at[pl.ds(0, C * KNN)]], buf, sem
        ).wait()

    def segsum(buf, g):
        def node(n, c2):
            for j in range(H // 16):
                sl = pl.ds(j * 16, 16)
                s = buf[n * KNN, sl]
                for k in range(1, KNN):
                    s = s + buf[n * KNN + k, sl]
                acc_v[n, sl] = s
            return c2

        lax.fori_loop(0, C, node, 0)
        pltpu.sync_copy(acc_v, out_hbm.at[pl.ds(base + g * C, C)])

    start(0, buf0, sem0)

    def pair(gg, carry):
        g0 = gg * 2
        start(g0 + 1, buf1, sem1)
        wait(buf0, sem0)
        segsum(buf0, g0)

        @pl.when(gg < NCHUNK // 2 - 1)
        def _():
            start(g0 + 2, buf0, sem0)

        wait(buf1, sem1)
        segsum(buf1, g0 + 1)
        return carry

    lax.fori_loop(0, NCHUNK // 2, pair, 0)


@functools.cache
def _make_sc_gather():
    return pl.kernel(
        _sc_gather_body,
        mesh=plsc.VectorSubcoreMesh(core_axis_name="c", subcore_axis_name="s"),
        out_type=jax.ShapeDtypeStruct((N, H), jnp.float32),
        scratch_types=[
            pltpu.VMEM((NPW * KNN,), jnp.int32),
            pltpu.VMEM((C * KNN, H), jnp.float32),
            pltpu.VMEM((C * KNN, H), jnp.float32),
            pltpu.VMEM((C, H), jnp.float32),
            pltpu.SemaphoreType.DMA,
            pltpu.SemaphoreType.DMA,
        ],
    )


def _sc_gather(keys, table):
    return _make_sc_gather()(keys, table)


def _stats_body(msg_ref, s_ref, bias_ref, pre_ref, ps_ref, pq_ref):
    pre = msg_ref[...] + s_ref[...] + bias_ref[...]
    pre_ref[...] = pre
    ps_ref[0] = jnp.sum(pre, axis=0, keepdims=True)
    pq_ref[0] = jnp.sum(pre * pre, axis=0, keepdims=True)


def _stats(msg, hall2d, bias):
    return pl.pallas_call(
        _stats_body,
        grid=(NBLK,),
        in_specs=[
            pl.BlockSpec((H, H), lambda i: (i, 0)),
            pl.BlockSpec((H, H), lambda i: (NUM_REL * NBLK + i, 0)),
            pl.BlockSpec((1, H), lambda i: (0, 0)),
        ],
        out_specs=[
            pl.BlockSpec((H, H), lambda i: (i, 0)),
            pl.BlockSpec((1, 1, H), lambda i: (i, 0, 0)),
            pl.BlockSpec((1, 1, H), lambda i: (i, 0, 0)),
        ],
        out_shape=[
            jax.ShapeDtypeStruct((N, H), jnp.float32),
            jax.ShapeDtypeStruct((NBLK, 1, H), jnp.float32),
            jax.ShapeDtypeStruct((NBLK, 1, H), jnp.float32),
        ],
    )(msg, hall2d, bias)


def _norm_body(pre_ref, h_ref, ps_ref, pq_ref, g_ref, b_ref, out_ref, outb_ref):
    s = jnp.sum(ps_ref[:, 0, :], axis=0, keepdims=True)
    q = jnp.sum(pq_ref[:, 0, :], axis=0, keepdims=True)
    mean = s * (1.0 / N)
    var = q * (1.0 / N) - mean * mean
    inv = lax.rsqrt(var + 1e-5)
    x = (pre_ref[...] - mean) * inv * g_ref[...] + b_ref[...]
    hn = jnp.maximum(x, 0.0) + h_ref[...]
    out_ref[...] = hn
    outb_ref[...] = hn.astype(jnp.bfloat16)


def _norm(pre, h, ps, pq, gamma, beta):
    return pl.pallas_call(
        _norm_body,
        grid=(NBLK,),
        in_specs=[
            pl.BlockSpec((H, H), lambda i: (i, 0)),
            pl.BlockSpec((H, H), lambda i: (i, 0)),
            pl.BlockSpec((NBLK, 1, H), lambda i: (0, 0, 0)),
            pl.BlockSpec((NBLK, 1, H), lambda i: (0, 0, 0)),
            pl.BlockSpec((1, H), lambda i: (0, 0)),
            pl.BlockSpec((1, H), lambda i: (0, 0)),
        ],
        out_specs=[
            pl.BlockSpec((H, H), lambda i: (i, 0)),
            pl.BlockSpec((H, H), lambda i: (i, 0)),
        ],
        out_shape=[
            jax.ShapeDtypeStruct((N, H), jnp.float32),
            jax.ShapeDtypeStruct((N, H), jnp.bfloat16),
        ],
    )(pre, h, ps, pq, gamma, beta)


def kernel(n_coords, ca_coords, c_coords, W0, b0, Wl, bl, Wsl, bsl, gamma, beta):
    ca = ca_coords.reshape(N, 3)
    cap = jnp.pad(ca, ((0, 0), (0, 5)))
    caT = jnp.pad(jnp.transpose(ca_coords, (0, 2, 1)),
                  ((0, 0), (0, 5), (0, 0)))

    keys4 = _knn_keys(cap, caT)
    keys = keys4.reshape(N, 16)[:, :KNN].reshape(-1)

    w0p = jnp.pad(W0, ((0, 5), (0, 0)))
    h, hb = _proj(cap, w0p, b0.reshape(1, H))

    wl3b = jnp.concatenate(
        [Wl.reshape(LAYERS, NUM_REL, H, H), Wsl[:, None]], axis=1
    ).astype(jnp.bfloat16)

    for i in range(LAYERS):
        hall2d = _hall(hb, wl3b[i]).reshape((NUM_REL + 1) * N, H)
        msg = _sc_gather(keys, hall2d)
        bias = (bl[i] + bsl[i]).reshape(1, H)
        pre, ps, pq = _stats(msg, hall2d, bias)
        h, hb = _norm(pre, h, ps, pq,
                      gamma[i].reshape(1, H), beta[i].reshape(1, H))

    return h.reshape(B, L, H)


# speedup vs baseline: 7.2727x; 1.0820x over previous
import functools

import jax
import jax.numpy as jnp
from jax import lax
from jax.experimental import pallas as pl
from jax.experimental.pallas import tpu as pltpu
from jax.experimental.pallas import tpu_sc as plsc

B = 4
L = 2048
N = B * L
H = 512
NUM_REL = 7
KNN = 10
LAYERS = 4

R = 256
NBLK = N // H

NC = 2
NS = 16
NW = NC * NS
NPW = N // NW
C = 8
NCHUNK = NPW // C


def _knn_body(rows_ref, cols_ref, key_ref):
    b = pl.program_id(0)
    rb = pl.program_id(1)
    r0 = rb * R
    a = rows_ref[...]
    cm = cols_ref[0, :, :]
    d2 = (a[:, 0:1] - cm[0:1, :]) ** 2
    d2 = d2 + (a[:, 1:2] - cm[1:2, :]) ** 2
    d2 = d2 + (a[:, 2:3] - cm[2:3, :]) ** 2
    col = lax.broadcasted_iota(jnp.int32, (R, L), 1)
    row2 = r0 + lax.broadcasted_iota(jnp.int32, (R, 1), 0)
    d2 = jnp.where(col == row2, d2 + 1e9, d2)

    kiota = lax.broadcasted_iota(jnp.int32, (R, 16), 1)
    kacc = jnp.zeros((R, 16), jnp.int32)
    big = jnp.float32(2e9)
    for k in range(KNN):
        m = jnp.min(d2, axis=1, keepdims=True)
        idx = jnp.min(jnp.where(d2 == m, col, N), axis=1,
                      keepdims=True)
        rel = jnp.clip(idx - row2, -3, 3) + 3
        key = rel * N + b * L + idx
        kacc = jnp.where(kiota == k, key, kacc)
        d2 = jnp.where(col == idx, big, d2)
    key_ref[0, :, :] = kacc


def _knn_keys(cap, caT):
    return pl.pallas_call(
        _knn_body,
        grid=(B, L // R),
        in_specs=[
            pl.BlockSpec((R, 8), lambda b, rb: (b * (L // R) + rb, 0)),
            pl.BlockSpec((1, 8, L), lambda b, rb: (b, 0, 0)),
        ],
        out_specs=pl.BlockSpec((1, R, 16), lambda b, rb: (b * (L // R) + rb, 0, 0)),
        out_shape=jax.ShapeDtypeStruct((B * (L // R), R, 16), jnp.int32),
    )(cap, caT)


def _proj_body(pos_ref, w_ref, b_ref, out_ref, outb_ref):
    h0 = (
        jnp.dot(pos_ref[...], w_ref[...], preferred_element_type=jnp.float32)
        + b_ref[...]
    )
    out_ref[...] = h0
    outb_ref[...] = h0.astype(jnp.bfloat16)


def _proj(posp, w0p, b0r):
    return pl.pallas_call(
        _proj_body,
        grid=(NBLK,),
        in_specs=[
            pl.BlockSpec((H, 8), lambda i: (i, 0)),
            pl.BlockSpec((8, H), lambda i: (0, 0)),
            pl.BlockSpec((1, H), lambda i: (0, 0)),
        ],
        out_specs=[
            pl.BlockSpec((H, H), lambda i: (i, 0)),
            pl.BlockSpec((H, H), lambda i: (i, 0)),
        ],
        out_shape=[
            jax.ShapeDtypeStruct((N, H), jnp.float32),
            jax.ShapeDtypeStruct((N, H), jnp.bfloat16),
        ],
    )(posp, w0p, b0r)


H2 = H // 2


def _rne_bf16_bits(y):
    u = jax.lax.bitcast_convert_type(y, jnp.int32)
    return (u + 0x7FFF + ((u >> 16) & 1)) >> 16


def _hall_body(hb_ref, w_ref, out_ref):
    y_lo = jnp.dot(hb_ref[...], w_ref[0, :, :H2],
                   preferred_element_type=jnp.float32)
    y_hi = jnp.dot(hb_ref[...], w_ref[0, :, H2:],
                   preferred_element_type=jnp.float32)
    out_ref[0] = (_rne_bf16_bits(y_lo) & 0xFFFF) | (_rne_bf16_bits(y_hi) << 16)


def _hall(hb, wl3b):
    return pl.pallas_call(
        _hall_body,
        grid=(NUM_REL + 1,),
        in_specs=[
            pl.BlockSpec((N, H), lambda r: (0, 0)),
            pl.BlockSpec((1, H, H), lambda r: (r, 0, 0)),
        ],
        out_specs=pl.BlockSpec((1, N, H2), lambda r: (r, 0, 0)),
        out_shape=jax.ShapeDtypeStruct((NUM_REL + 1, N, H2), jnp.int32),
    )(hb, wl3b)


def _sc_gather_body(keys_hbm, table_hbm, out_hbm, idx_v, buf0, buf1, acc_v,
                    sem0, sem1):
    wid = lax.axis_index("s") * NC + lax.axis_index("c")
    base = wid * NPW
    pltpu.sync_copy(keys_hbm.at[pl.ds(wid * NPW * KNN, NPW * KNN)], idx_v)

    def start(g, buf, sem):
        return pltpu.async_copy(
            table_hbm.at[idx_v.at[pl.ds(g * C * KNN, C * KNN)]], buf, sem
        )

    def wait(buf, sem):
        pltpu.make_async_copy(
            table_hbm.at[idx_v.at[pl.ds(0, C * KNN)]], buf, sem
        ).wait()

    def segsum(buf, g):
        def node(n, c2):
            for j in range(H2 // 16):
                sl = pl.ds(j * 16, 16)
                vi = buf[n * KNN, sl]
                lo = plsc.bitcast(vi << 16, jnp.float32)
                hi = plsc.bitcast(vi & jnp.int32(-65536), jnp.float32)
                for k in range(1, KNN):
                    vi = buf[n * KNN + k, sl]
                    lo = lo + plsc.bitcast(vi << 16, jnp.float32)
                    hi = hi + plsc.bitcast(vi & jnp.int32(-65536), jnp.float32)
                acc_v[n, sl] = lo
                acc_v[n, pl.ds(H2 + j * 16, 16)] = hi
            return c2

        lax.fori_loop(0, C, node, 0)
        pltpu.sync_copy(acc_v, out_hbm.at[pl.ds(base + g * C, C)])

    start(0, buf0, sem0)

    def pair(gg, carry):
        g0 = gg * 2
        start(g0 + 1, buf1, sem1)
        wait(buf0, sem0)
        segsum(buf0, g0)

        @pl.when(gg < NCHUNK // 2 - 1)
        def _():
            start(g0 + 2, buf0, sem0)

        wait(buf1, sem1)
        segsum(buf1, g0 + 1)
        return carry

    lax.fori_loop(0, NCHUNK // 2, pair, 0)


@functools.cache
def _make_sc_gather():
    return pl.kernel(
        _sc_gather_body,
        mesh=plsc.VectorSubcoreMesh(core_axis_name="c", subcore_axis_name="s"),
        compiler_params=pltpu.CompilerParams(needs_layout_passes=False),
        out_type=jax.ShapeDtypeStruct((N, H), jnp.float32),
        scratch_types=[
            pltpu.VMEM((NPW * KNN,), jnp.int32),
            pltpu.VMEM((C * KNN, H2), jnp.int32),
            pltpu.VMEM((C * KNN, H2), jnp.int32),
            pltpu.VMEM((C, H), jnp.float32),
            pltpu.SemaphoreType.DMA,
            pltpu.SemaphoreType.DMA,
        ],
    )


def _sc_gather(keys, table):
    return _make_sc_gather()(keys, table)


def _stats_body(msg_ref, s_ref, bias_ref, pre_ref, ps_ref, pq_ref):
    s32 = s_ref[...]
    s_lo = jax.lax.bitcast_convert_type(s32 << 16, jnp.float32)
    s_hi = jax.lax.bitcast_convert_type(s32 & jnp.int32(-65536), jnp.float32)
    s = jnp.concatenate([s_lo, s_hi], axis=1)
    pre = msg_ref[...] + s + bias_ref[...]
    pre_ref[...] = pre
    ps_ref[0] = jnp.sum(pre, axis=0, keepdims=True)
    pq_ref[0] = jnp.sum(pre * pre, axis=0, keepdims=True)


def _stats(msg, hall2d, bias):
    return pl.pallas_call(
        _stats_body,
        grid=(NBLK,),
        in_specs=[
            pl.BlockSpec((H, H), lambda i: (i, 0)),
            pl.BlockSpec((H, H2), lambda i: (NUM_REL * NBLK + i, 0)),
            pl.BlockSpec((1, H), lambda i: (0, 0)),
        ],
        out_specs=[
            pl.BlockSpec((H, H), lambda i: (i, 0)),
            pl.BlockSpec((1, 1, H), lambda i: (i, 0, 0)),
            pl.BlockSpec((1, 1, H), lambda i: (i, 0, 0)),
        ],
        out_shape=[
            jax.ShapeDtypeStruct((N, H), jnp.float32),
            jax.ShapeDtypeStruct((NBLK, 1, H), jnp.float32),
            jax.ShapeDtypeStruct((NBLK, 1, H), jnp.float32),
        ],
    )(msg, hall2d, bias)


def _norm_body(pre_ref, h_ref, ps_ref, pq_ref, g_ref, b_ref, out_ref, outb_ref):
    s = jnp.sum(ps_ref[:, 0, :], axis=0, keepdims=True)
    q = jnp.sum(pq_ref[:, 0, :], axis=0, keepdims=True)
    mean = s * (1.0 / N)
    var = q * (1.0 / N) - mean * mean
    inv = lax.rsqrt(var + 1e-5)
    x = (pre_ref[...] - mean) * inv * g_ref[...] + b_ref[...]
    hn = jnp.maximum(x, 0.0) + h_ref[...]
    out_ref[...] = hn
    outb_ref[...] = hn.astype(jnp.bfloat16)


def _norm(pre, h, ps, pq, gamma, beta):
    return pl.pallas_call(
        _norm_body,
        grid=(NBLK,),
        in_specs=[
            pl.BlockSpec((H, H), lambda i: (i, 0)),
            pl.BlockSpec((H, H), lambda i: (i, 0)),
            pl.BlockSpec((NBLK, 1, H), lambda i: (0, 0, 0)),
            pl.BlockSpec((NBLK, 1, H), lambda i: (0, 0, 0)),
            pl.BlockSpec((1, H), lambda i: (0, 0)),
            pl.BlockSpec((1, H), lambda i: (0, 0)),
        ],
        out_specs=[
            pl.BlockSpec((H, H), lambda i: (i, 0)),
            pl.BlockSpec((H, H), lambda i: (i, 0)),
        ],
        out_shape=[
            jax.ShapeDtypeStruct((N, H), jnp.float32),
            jax.ShapeDtypeStruct((N, H), jnp.bfloat16),
        ],
    )(pre, h, ps, pq, gamma, beta)


def kernel(n_coords, ca_coords, c_coords, W0, b0, Wl, bl, Wsl, bsl, gamma, beta):
    ca = ca_coords.reshape(N, 3)
    cap = jnp.pad(ca, ((0, 0), (0, 5)))
    caT = jnp.pad(jnp.transpose(ca_coords, (0, 2, 1)),
                  ((0, 0), (0, 5), (0, 0)))

    keys4 = _knn_keys(cap, caT)
    keys = keys4.reshape(N, 16)[:, :KNN].reshape(-1)

    u = jnp.arange(H)
    wperm = 32 * ((u % H2) // 16) + 16 * (u // H2) + (u % 16)
    invw = jnp.argsort(wperm)

    w0p = jnp.pad(W0, ((0, 5), (0, 0)))[:, wperm]
    h, hb = _proj(cap, w0p, b0[wperm].reshape(1, H))

    wl3b = jnp.concatenate(
        [Wl.reshape(LAYERS, NUM_REL, H, H), Wsl[:, None]], axis=1
    )[:, :, wperm, :][:, :, :, wperm].astype(jnp.bfloat16)

    for i in range(LAYERS):
        hall2d = _hall(hb, wl3b[i]).reshape((NUM_REL + 1) * N, H2)
        msg = _sc_gather(keys, hall2d)
        bias = (bl[i] + bsl[i])[wperm].reshape(1, H)
        pre, ps, pq = _stats(msg, hall2d, bias)
        h, hb = _norm(pre, h, ps, pq,
                      gamma[i][wperm].reshape(1, H),
                      beta[i][wperm].reshape(1, H))

    return h[:, invw].reshape(B, L, H)


# speedup vs baseline: 7.6559x; 1.0527x over previous
import functools

import jax
import jax.numpy as jnp
from jax import lax
from jax.experimental import pallas as pl
from jax.experimental.pallas import tpu as pltpu
from jax.experimental.pallas import tpu_sc as plsc

B = 4
L = 2048
N = B * L
H = 512
NUM_REL = 7
KNN = 10
LAYERS = 4

R = 256
NBLK = N // H

NC = 2
NS = 16
NW = NC * NS
NPW = N // NW
C = 8
NCHUNK = NPW // C


def _knn_body(rows_ref, cols_ref, key_ref):
    b = pl.program_id(0)
    rb = pl.program_id(1)
    r0 = rb * R
    a = rows_ref[...]
    cm = cols_ref[0, :, :]
    d2 = (a[:, 0:1] - cm[0:1, :]) ** 2
    d2 = d2 + (a[:, 1:2] - cm[1:2, :]) ** 2
    d2 = d2 + (a[:, 2:3] - cm[2:3, :]) ** 2
    col = lax.broadcasted_iota(jnp.int32, (R, L), 1)
    row2 = r0 + lax.broadcasted_iota(jnp.int32, (R, 1), 0)
    d2 = jnp.where(col == row2, d2 + 1e9, d2)

    kiota = lax.broadcasted_iota(jnp.int32, (R, 16), 1)
    kacc = jnp.zeros((R, 16), jnp.int32)
    big = jnp.float32(2e9)
    for k in range(KNN):
        m = jnp.min(d2, axis=1, keepdims=True)
        idx = jnp.min(jnp.where(d2 == m, col, N), axis=1,
                      keepdims=True)
        rel = jnp.clip(idx - row2, -3, 3) + 3
        key = rel * N + b * L + idx
        kacc = jnp.where(kiota == k, key, kacc)
        d2 = jnp.where(col == idx, big, d2)
    key_ref[0, :, :] = kacc


def _knn_keys(cap, caT):
    return pl.pallas_call(
        _knn_body,
        grid=(B, L // R),
        in_specs=[
            pl.BlockSpec((R, 8), lambda b, rb: (b * (L // R) + rb, 0)),
            pl.BlockSpec((1, 8, L), lambda b, rb: (b, 0, 0)),
        ],
        out_specs=pl.BlockSpec((1, R, 16), lambda b, rb: (b * (L // R) + rb, 0, 0)),
        out_shape=jax.ShapeDtypeStruct((B * (L // R), R, 16), jnp.int32),
    )(cap, caT)


def _proj_body(pos_ref, w_ref, b_ref, out_ref, outb_ref):
    h0 = (
        jnp.dot(pos_ref[...], w_ref[...], preferred_element_type=jnp.float32)
        + b_ref[...]
    )
    out_ref[...] = h0
    outb_ref[...] = h0.astype(jnp.bfloat16)


def _proj(posp, w0p, b0r):
    return pl.pallas_call(
        _proj_body,
        grid=(NBLK,),
        in_specs=[
            pl.BlockSpec((H, 8), lambda i: (i, 0)),
            pl.BlockSpec((8, H), lambda i: (0, 0)),
            pl.BlockSpec((1, H), lambda i: (0, 0)),
        ],
        out_specs=[
            pl.BlockSpec((H, H), lambda i: (i, 0)),
            pl.BlockSpec((H, H), lambda i: (i, 0)),
        ],
        out_shape=[
            jax.ShapeDtypeStruct((N, H), jnp.float32),
            jax.ShapeDtypeStruct((N, H), jnp.bfloat16),
        ],
    )(posp, w0p, b0r)


H2 = H // 2


def _hall_body(hb_ref, w_ref, out_ref):
    y_lo = jnp.dot(hb_ref[...], w_ref[0, :, :H2],
                   preferred_element_type=jnp.float32)
    y_hi = jnp.dot(hb_ref[...], w_ref[0, :, H2:],
                   preferred_element_type=jnp.float32)
    u_lo = jax.lax.bitcast_convert_type(y_lo, jnp.int32)
    u_hi = jax.lax.bitcast_convert_type(y_hi, jnp.int32)
    out_ref[0] = ((u_lo >> 16) & 0xFFFF) | (u_hi & jnp.int32(-65536))


def _hall(hb, wl3b):
    return pl.pallas_call(
        _hall_body,
        grid=(NUM_REL + 1,),
        in_specs=[
            pl.BlockSpec((N, H), lambda r: (0, 0)),
            pl.BlockSpec((1, H, H), lambda r: (r, 0, 0)),
        ],
        out_specs=pl.BlockSpec((1, N, H2), lambda r: (r, 0, 0)),
        out_shape=jax.ShapeDtypeStruct((NUM_REL + 1, N, H2), jnp.int32),
    )(hb, wl3b)


def _sc_gather_body(keys_hbm, table_hbm, out_hbm, idx_v, buf0, buf1, acc_v,
                    sem0, sem1):
    wid = lax.axis_index("s") * NC + lax.axis_index("c")
    base = wid * NPW
    pltpu.sync_copy(keys_hbm.at[pl.ds(wid * NPW * KNN, NPW * KNN)], idx_v)

    def start(g, buf, sem):
        return pltpu.async_copy(
            table_hbm.at[idx_v.at[pl.ds(g * C * KNN, C * KNN)]], buf, sem
        )

    def wait(buf, sem):
        pltpu.make_async_copy(
            table_hbm.at[idx_v.at[pl.ds(0, C * KNN)]], buf, sem
        ).wait()

    def segsum(buf, g):
        def node(n, c2):
            for j in range(H2 // 16):
                sl = pl.ds(j * 16, 16)
                vi = buf[n * KNN, sl]
                lo = plsc.bitcast(vi << 16, jnp.float32)
                hi = plsc.bitcast(vi & jnp.int32(-65536), jnp.float32)
                for k in range(1, KNN):
                    vi = buf[n * KNN + k, sl]
                    lo = lo + plsc.bitcast(vi << 16, jnp.float32)
                    hi = hi + plsc.bitcast(vi & jnp.int32(-65536), jnp.float32)
                acc_v[n, sl] = lo
                acc_v[n, pl.ds(H2 + j * 16, 16)] = hi
            return c2

        lax.fori_loop(0, C, node, 0)
        pltpu.sync_copy(acc_v, out_hbm.at[pl.ds(base + g * C, C)])

    start(0, buf0, sem0)

    def pair(gg, carry):
        g0 = gg * 2
        start(g0 + 1, buf1, sem1)
        wait(buf0, sem0)
        segsum(buf0, g0)

        @pl.when(gg < NCHUNK // 2 - 1)
        def _():
            start(g0 + 2, buf0, sem0)

        wait(buf1, sem1)
        segsum(buf1, g0 + 1)
        return carry

    lax.fori_loop(0, NCHUNK // 2, pair, 0)


@functools.cache
def _make_sc_gather():
    return pl.kernel(
        _sc_gather_body,
        mesh=plsc.VectorSubcoreMesh(core_axis_name="c", subcore_axis_name="s"),
        compiler_params=pltpu.CompilerParams(needs_layout_passes=False),
        out_type=jax.ShapeDtypeStruct((N, H), jnp.float32),
        scratch_types=[
            pltpu.VMEM((NPW * KNN,), jnp.int32),
            pltpu.VMEM((C * KNN, H2), jnp.int32),
            pltpu.VMEM((C * KNN, H2), jnp.int32),
            pltpu.VMEM((C, H), jnp.float32),
            pltpu.SemaphoreType.DMA,
            pltpu.SemaphoreType.DMA,
        ],
    )


def _sc_gather(keys, table):
    return _make_sc_gather()(keys, table)


def _decode_s(s32):
    s_lo = jax.lax.bitcast_convert_type(s32 << 16, jnp.float32)
    s_hi = jax.lax.bitcast_convert_type(s32 & jnp.int32(-65536), jnp.float32)
    return jnp.concatenate([s_lo, s_hi], axis=1)


def _stats_body(msg_ref, s_ref, bias_ref, ps_ref, pq_ref):
    pre = msg_ref[...] + _decode_s(s_ref[...]) + bias_ref[...]
    ps_ref[0] = jnp.sum(pre, axis=0, keepdims=True)
    pq_ref[0] = jnp.sum(pre * pre, axis=0, keepdims=True)


def _stats(msg, hall2d, bias):
    return pl.pallas_call(
        _stats_body,
        grid=(NBLK,),
        in_specs=[
            pl.BlockSpec((H, H), lambda i: (i, 0)),
            pl.BlockSpec((H, H2), lambda i: (NUM_REL * NBLK + i, 0)),
            pl.BlockSpec((1, H), lambda i: (0, 0)),
        ],
        out_specs=[
            pl.BlockSpec((1, 1, H), lambda i: (i, 0, 0)),
            pl.BlockSpec((1, 1, H), lambda i: (i, 0, 0)),
        ],
        out_shape=[
            jax.ShapeDtypeStruct((NBLK, 1, H), jnp.float32),
            jax.ShapeDtypeStruct((NBLK, 1, H), jnp.float32),
        ],
    )(msg, hall2d, bias)


def _norm_body(msg_ref, s_ref, bias_ref, h_ref, ps_ref, pq_ref, g_ref, b_ref,
               out_ref, outb_ref):
    s = jnp.sum(ps_ref[:, 0, :], axis=0, keepdims=True)
    q = jnp.sum(pq_ref[:, 0, :], axis=0, keepdims=True)
    mean = s * (1.0 / N)
    var = q * (1.0 / N) - mean * mean
    inv = lax.rsqrt(var + 1e-5)
    pre = msg_ref[...] + _decode_s(s_ref[...]) + bias_ref[...]
    x = (pre - mean) * inv * g_ref[...] + b_ref[...]
    hn = jnp.maximum(x, 0.0) + h_ref[...]
    out_ref[...] = hn
    outb_ref[...] = hn.astype(jnp.bfloat16)


def _norm(msg, hall2d, bias, h, ps, pq, gamma, beta):
    return pl.pallas_call(
        _norm_body,
        grid=(NBLK,),
        in_specs=[
            pl.BlockSpec((H, H), lambda i: (i, 0)),
            pl.BlockSpec((H, H2), lambda i: (NUM_REL * NBLK + i, 0)),
            pl.BlockSpec((1, H), lambda i: (0, 0)),
            pl.BlockSpec((H, H), lambda i: (i, 0)),
            pl.BlockSpec((NBLK, 1, H), lambda i: (0, 0, 0)),
            pl.BlockSpec((NBLK, 1, H), lambda i: (0, 0, 0)),
            pl.BlockSpec((1, H), lambda i: (0, 0)),
            pl.BlockSpec((1, H), lambda i: (0, 0)),
        ],
        out_specs=[
            pl.BlockSpec((H, H), lambda i: (i, 0)),
            pl.BlockSpec((H, H), lambda i: (i, 0)),
        ],
        out_shape=[
            jax.ShapeDtypeStruct((N, H), jnp.float32),
            jax.ShapeDtypeStruct((N, H), jnp.bfloat16),
        ],
    )(msg, hall2d, bias, h, ps, pq, gamma, beta)


def kernel(n_coords, ca_coords, c_coords, W0, b0, Wl, bl, Wsl, bsl, gamma, beta):
    ca = ca_coords.reshape(N, 3)
    cap = jnp.pad(ca, ((0, 0), (0, 5)))
    caT = jnp.pad(jnp.transpose(ca_coords, (0, 2, 1)),
                  ((0, 0), (0, 5), (0, 0)))

    keys4 = _knn_keys(cap, caT)
    keys = keys4.reshape(N, 16)[:, :KNN].reshape(-1)

    u = jnp.arange(H)
    wperm = 32 * ((u % H2) // 16) + 16 * (u // H2) + (u % 16)
    invw = jnp.argsort(wperm)

    w0p = jnp.pad(W0, ((0, 5), (0, 0)))[:, wperm]
    h, hb = _proj(cap, w0p, b0[wperm].reshape(1, H))

    wl3b = jnp.concatenate(
        [Wl.reshape(LAYERS, NUM_REL, H, H), Wsl[:, None]], axis=1
    )[:, :, wperm, :][:, :, :, wperm].astype(jnp.bfloat16)

    for i in range(LAYERS):
        hall2d = _hall(hb, wl3b[i]).reshape((NUM_REL + 1) * N, H2)
        msg = _sc_gather(keys, hall2d)
        bias = (bl[i] + bsl[i])[wperm].reshape(1, H)
        ps, pq = _stats(msg, hall2d, bias)
        h, hb = _norm(msg, hall2d, bias, h, ps, pq,
                      gamma[i][wperm].reshape(1, H),
                      beta[i][wperm].reshape(1, H))

    return h[:, invw].reshape(B, L, H)
